# Initial kernel scaffold; baseline (speedup 1.0000x reference)
import jax
import jax.numpy as jnp
from jax.experimental import pallas as pl


def kernel(x, edge_index, edge_attr, W1, b1, W2, b2):
    raise NotImplementedError("write your pallas kernel here")


# speedup vs baseline: 2.9218x; 2.9218x over previous
import functools

import jax
import jax.numpy as jnp
from jax import lax
from jax.experimental import pallas as pl
from jax.experimental.pallas import tpu as pltpu
from jax.experimental.pallas import tpu_sc as plsc

N_NODES = 10000
D = 128
E = 320000
DE = 16

NC = 2
NS = 16
NW = NC * NS
TOTAL_ROWS = 2 * E
PER_W = TOTAL_ROWS // NW
CHUNK = 80
STEPS = PER_W // CHUNK

BE = 2000
NBLK = E // BE


def _sc_gather_body(x_hbm, idx_hbm, out_hbm, idx_v, rows_v, sem):
    c = lax.axis_index("c")
    s = lax.axis_index("s")
    wid = s * NC + c
    base = wid * PER_W
    pltpu.sync_copy(idx_hbm.at[wid], idx_v)

    def body(j, carry):
        pltpu.async_copy(x_hbm.at[idx_v.at[j]], rows_v, sem).wait()
        pltpu.sync_copy(rows_v, out_hbm.at[pl.ds(base + j * CHUNK, CHUNK)])
        return carry

    lax.fori_loop(0, STEPS, body, 0)


@functools.cache
def _sc_gather():
    return functools.partial(
        pl.kernel,
        out_type=jax.ShapeDtypeStruct((TOTAL_ROWS, D), jnp.float32),
        mesh=plsc.VectorSubcoreMesh(core_axis_name="c", subcore_axis_name="s"),
        scratch_types=[
            pltpu.VMEM((STEPS, CHUNK), jnp.int32),
            pltpu.VMEM((CHUNK, D), jnp.float32),
            pltpu.SemaphoreType.DMA,
        ],
    )(_sc_gather_body)


def _mlp_body(xs_ref, xd_ref, ea_ref, w1a_ref, w1b_ref, w1c_ref, b1_ref,
              w2_ref, b2_ref, out_ref):
    z = jnp.dot(xs_ref[...], w1a_ref[...], preferred_element_type=jnp.float32)
    z = z + jnp.dot(xd_ref[...], w1b_ref[...], preferred_element_type=jnp.float32)
    z = z + jnp.dot(ea_ref[...], w1c_ref[...], preferred_element_type=jnp.float32)
    z = z + b1_ref[...]
    h = 0.5 * z * (1.0 + lax.erf(z * 0.7071067811865476))
    out_ref[...] = (
        jnp.dot(h, w2_ref[...], preferred_element_type=jnp.float32) + b2_ref[...]
    )


def _mlp(rows, edge_attr, W1a, W1b, W1c, b1, W2, b2):
    return pl.pallas_call(
        _mlp_body,
        grid=(NBLK,),
        in_specs=[
            pl.BlockSpec((BE, D), lambda i: (i, 0)),
            pl.BlockSpec((BE, D), lambda i: (i + NBLK, 0)),
            pl.BlockSpec((BE, DE), lambda i: (i, 0)),
            pl.BlockSpec((D, D), lambda i: (0, 0)),
            pl.BlockSpec((D, D), lambda i: (0, 0)),
            pl.BlockSpec((DE, D), lambda i: (0, 0)),
            pl.BlockSpec((1, D), lambda i: (0, 0)),
            pl.BlockSpec((D, D), lambda i: (0, 0)),
            pl.BlockSpec((1, D), lambda i: (0, 0)),
        ],
        out_specs=pl.BlockSpec((BE, D), lambda i: (i, 0)),
        out_shape=jax.ShapeDtypeStruct((E, D), jnp.float32),
    )(rows, rows, edge_attr, W1a, W1b, W1c, b1, W2, b2)


def kernel(x, edge_index, edge_attr, W1, b1, W2, b2):
    idx = edge_index.astype(jnp.int32).reshape(NW, STEPS, CHUNK)
    rows = _sc_gather()(x, idx)
    W1a = W1[:D]
    W1b = W1[D:2 * D]
    W1c = W1[2 * D:]
    return _mlp(rows, edge_attr, W1a, W1b, W1c,
                b1.reshape(1, D), W2, b2.reshape(1, D))


# speedup vs baseline: 3.6465x; 1.2480x over previous
import functools

import jax
import jax.numpy as jnp
from jax import lax
from jax.experimental import pallas as pl
from jax.experimental.pallas import tpu as pltpu
from jax.experimental.pallas import tpu_sc as plsc

N_NODES = 10000
D = 128
E = 320000
DE = 16

NC = 2
NS = 16
NW = NC * NS
TOTAL_ROWS = 2 * E
PER_W = TOTAL_ROWS // NW
CHUNK = 100
STEPS = PER_W // CHUNK
GRP = 4
GROUP_ROWS = GRP * CHUNK
NGRP = STEPS // GRP
OUTER = NGRP // 2

BE = 2000
NBLK = E // BE


def _sc_gather_body(x_hbm, idx_hbm, out_hbm, idx_v, rows_v, gs0, gs1, ws0, ws1):
    c = lax.axis_index("c")
    s = lax.axis_index("s")
    wid = s * NC + c
    base = wid * PER_W
    pltpu.sync_copy(idx_hbm.at[wid], idx_v)
    g_sems = (gs0, gs1)
    w_sems = (ws0, ws1)

    def gather(g, b, k):
        return (x_hbm.at[idx_v.at[g * GRP + k]],
                rows_v.at[b, pl.ds(k * CHUNK, CHUNK)],
                g_sems[b])

    def write(g, b):
        return (rows_v.at[b],
                out_hbm.at[pl.ds(base + g * GROUP_ROWS, GROUP_ROWS)],
                w_sems[b])

    def outer(t, carry):
        for b in (0, 1):
            g = 2 * t + b

            @pl.when(t > 0)
            def _(b=b, g=g):
                pltpu.make_async_copy(*write(g - 2, b)).wait()

            for k in range(GRP):
                pltpu.async_copy(*gather(g, b, k))
        for b in (0, 1):
            g = 2 * t + b
            for k in range(GRP):
                pltpu.make_async_copy(*gather(g, b, k)).wait()
            pltpu.async_copy(*write(g, b))
        return carry

    lax.fori_loop(0, OUTER, outer, 0)
    for b in (0, 1):
        pltpu.make_async_copy(*write(2 * (OUTER - 1) + b, b)).wait()


@functools.cache
def _sc_gather():
    return functools.partial(
        pl.kernel,
        out_type=jax.ShapeDtypeStruct((TOTAL_ROWS, D), jnp.float32),
        mesh=plsc.VectorSubcoreMesh(core_axis_name="c", subcore_axis_name="s"),
        scratch_types=[
            pltpu.VMEM((STEPS, CHUNK), jnp.int32),
            pltpu.VMEM((2, GROUP_ROWS, D), jnp.float32),
            pltpu.SemaphoreType.DMA,
            pltpu.SemaphoreType.DMA,
            pltpu.SemaphoreType.DMA,
            pltpu.SemaphoreType.DMA,
        ],
    )(_sc_gather_body)


def _mlp_body(xs_ref, xd_ref, ea_ref, w1a_ref, w1b_ref, w1c_ref, b1_ref,
              w2_ref, b2_ref, out_ref):
    z = jnp.dot(xs_ref[...], w1a_ref[...], preferred_element_type=jnp.float32)
    z = z + jnp.dot(xd_ref[...], w1b_ref[...], preferred_element_type=jnp.float32)
    z = z + jnp.dot(ea_ref[...], w1c_ref[...], preferred_element_type=jnp.float32)
    z = z + b1_ref[...]
    h = 0.5 * z * (1.0 + lax.erf(z * 0.7071067811865476))
    out_ref[...] = (
        jnp.dot(h, w2_ref[...], preferred_element_type=jnp.float32) + b2_ref[...]
    )


def _mlp(rows, edge_attr, W1a, W1b, W1c, b1, W2, b2):
    return pl.pallas_call(
        _mlp_body,
        grid=(NBLK,),
        in_specs=[
            pl.BlockSpec((BE, D), lambda i: (i, 0)),
            pl.BlockSpec((BE, D), lambda i: (i + NBLK, 0)),
            pl.BlockSpec((BE, DE), lambda i: (i, 0)),
            pl.BlockSpec((D, D), lambda i: (0, 0)),
            pl.BlockSpec((D, D), lambda i: (0, 0)),
            pl.BlockSpec((DE, D), lambda i: (0, 0)),
            pl.BlockSpec((1, D), lambda i: (0, 0)),
            pl.BlockSpec((D, D), lambda i: (0, 0)),
            pl.BlockSpec((1, D), lambda i: (0, 0)),
        ],
        out_specs=pl.BlockSpec((BE, D), lambda i: (i, 0)),
        out_shape=jax.ShapeDtypeStruct((E, D), jnp.float32),
    )(rows, rows, edge_attr, W1a, W1b, W1c, b1, W2, b2)


def kernel(x, edge_index, edge_attr, W1, b1, W2, b2):
    idx = edge_index.astype(jnp.int32).reshape(NW, STEPS, CHUNK)
    rows = _sc_gather()(x, idx)
    W1a = W1[:D]
    W1b = W1[D:2 * D]
    W1c = W1[2 * D:]
    return _mlp(rows, edge_attr, W1a, W1b, W1c,
                b1.reshape(1, D), W2, b2.reshape(1, D))


# speedup vs baseline: 3.8286x; 1.0499x over previous
import functools

import jax
import jax.numpy as jnp
from jax import lax
from jax.experimental import pallas as pl
from jax.experimental.pallas import tpu as pltpu
from jax.experimental.pallas import tpu_sc as plsc

N_NODES = 10000
D = 128
E = 320000
DE = 16

S = 2
EH = E // S

NC = 2
NS = 16
NW = NC * NS
ROWS_S = 2 * EH
PER_W = ROWS_S // NW
CHUNK = 100
STEPS = PER_W // CHUNK
GRP = 2
GROUP_ROWS = GRP * CHUNK
NGRP = STEPS // GRP
OUTER = NGRP // 2

BE = 2000
NBLK_H = EH // BE


def _sc_gather_body(x_hbm, idx_hbm, out_hbm, idx_v, rows_v, gs0, gs1, ws0, ws1):
    c = lax.axis_index("c")
    s = lax.axis_index("s")
    wid = s * NC + c
    base = wid * PER_W
    pltpu.sync_copy(idx_hbm.at[wid], idx_v)
    g_sems = (gs0, gs1)
    w_sems = (ws0, ws1)

    def gather(g, b, k):
        return (x_hbm.at[idx_v.at[g * GRP + k]],
                rows_v.at[b, pl.ds(k * CHUNK, CHUNK)],
                g_sems[b])

    def write(g, b):
        return (rows_v.at[b],
                out_hbm.at[pl.ds(base + g * GROUP_ROWS, GROUP_ROWS)],
                w_sems[b])

    def outer(t, carry):
        for b in (0, 1):
            g = 2 * t + b

            @pl.when(t > 0)
            def _(b=b, g=g):
                pltpu.make_async_copy(*write(g - 2, b)).wait()

            for k in range(GRP):
                pltpu.async_copy(*gather(g, b, k))
        for b in (0, 1):
            g = 2 * t + b
            for k in range(GRP):
                pltpu.make_async_copy(*gather(g, b, k)).wait()
            pltpu.async_copy(*write(g, b))
        return carry

    lax.fori_loop(0, OUTER, outer, 0)
    for b in (0, 1):
        pltpu.make_async_copy(*write(2 * (OUTER - 1) + b, b)).wait()


@functools.cache
def _sc_gather():
    return functools.partial(
        pl.kernel,
        out_type=jax.ShapeDtypeStruct((ROWS_S, D), jnp.float32),
        mesh=plsc.VectorSubcoreMesh(core_axis_name="c", subcore_axis_name="s"),
        scratch_types=[
            pltpu.VMEM((STEPS, CHUNK), jnp.int32),
            pltpu.VMEM((2, GROUP_ROWS, D), jnp.float32),
            pltpu.SemaphoreType.DMA,
            pltpu.SemaphoreType.DMA,
            pltpu.SemaphoreType.DMA,
            pltpu.SemaphoreType.DMA,
        ],
    )(_sc_gather_body)


def _mlp_compute(xs_ref, xd_ref, ea_ref, w1a_ref, w1b_ref, w1c_ref, b1_ref,
                 w2_ref, b2_ref, out_ref):
    z = jnp.dot(xs_ref[...], w1a_ref[...], preferred_element_type=jnp.float32)
    z = z + jnp.dot(xd_ref[...], w1b_ref[...], preferred_element_type=jnp.float32)
    z = z + jnp.dot(ea_ref[...], w1c_ref[...], preferred_element_type=jnp.float32)
    z = z + b1_ref[...]
    h = 0.5 * z * (1.0 + lax.erf(z * 0.7071067811865476))
    out_ref[...] = (
        jnp.dot(h, w2_ref[...], preferred_element_type=jnp.float32) + b2_ref[...]
    )


def _mlp_body_first(*refs):
    _mlp_compute(*refs)


def _mlp_body_second(prev_ref, *refs):
    del prev_ref
    _mlp_compute(*refs)


_W_SPECS = [
    pl.BlockSpec((D, D), lambda i: (0, 0)),
    pl.BlockSpec((D, D), lambda i: (0, 0)),
    pl.BlockSpec((DE, D), lambda i: (0, 0)),
    pl.BlockSpec((1, D), lambda i: (0, 0)),
    pl.BlockSpec((D, D), lambda i: (0, 0)),
    pl.BlockSpec((1, D), lambda i: (0, 0)),
]


def _mlp_slice(h, rows, edge_attr, weights, prev_out=None):
    off = h * NBLK_H
    in_specs = [
        pl.BlockSpec((BE, D), lambda i: (i, 0)),
        pl.BlockSpec((BE, D), lambda i: (i + NBLK_H, 0)),
        pl.BlockSpec((BE, DE), lambda i, off=off: (i + off, 0)),
    ] + _W_SPECS
    out_spec = pl.BlockSpec((BE, D), lambda i, off=off: (i + off, 0))
    args = (rows, rows, edge_attr) + weights
    if prev_out is None:
        return pl.pallas_call(
            _mlp_body_first,
            grid=(NBLK_H,),
            in_specs=in_specs,
            out_specs=out_spec,
            out_shape=jax.ShapeDtypeStruct((E, D), jnp.float32),
        )(*args)
    return pl.pallas_call(
        _mlp_body_second,
        grid=(NBLK_H,),
        in_specs=[pl.BlockSpec(memory_space=pl.ANY)] + in_specs,
        out_specs=out_spec,
        out_shape=jax.ShapeDtypeStruct((E, D), jnp.float32),
        input_output_aliases={0: 0},
    )(prev_out, *args)


def kernel(x, edge_index, edge_attr, W1, b1, W2, b2):
    src = edge_index[0].astype(jnp.int32)
    dst = edge_index[1].astype(jnp.int32)
    weights = (W1[:D], W1[D:2 * D], W1[2 * D:],
               b1.reshape(1, D), W2, b2.reshape(1, D))
    sc = _sc_gather()
    rows = []
    for h in range(S):
        idx = jnp.concatenate(
            [src[h * EH:(h + 1) * EH], dst[h * EH:(h + 1) * EH]]
        ).reshape(NW, STEPS, CHUNK)
        rows.append(sc(x, idx))
    out = _mlp_slice(0, rows[0], edge_attr, weights)
    for h in range(1, S):
        out = _mlp_slice(h, rows[h], edge_attr, weights, prev_out=out)
    return out


# speedup vs baseline: 4.8672x; 1.2713x over previous
import functools

import jax
import jax.numpy as jnp
from jax import lax
from jax.experimental import pallas as pl
from jax.experimental.pallas import tpu as pltpu
from jax.experimental.pallas import tpu_sc as plsc

N_NODES = 10000
D = 128
E = 320000
DE = 16

S = 2
EH = E // S

NC = 2
NS = 16
NW = NC * NS
ROWS_S = 2 * EH
PER_W = ROWS_S // NW
CHUNK = 100
STEPS = PER_W // CHUNK
GRP = 2
GROUP_ROWS = GRP * CHUNK
NGRP = STEPS // GRP
OUTER = NGRP // 2

BE = 3200
NBLK_H = EH // BE


def _sc_gather_body(x_hbm, idx_hbm, out_hbm, idx_v, rows_v, gs0, gs1, ws0, ws1):
    c = lax.axis_index("c")
    s = lax.axis_index("s")
    wid = s * NC + c
    base = wid * PER_W
    pltpu.sync_copy(idx_hbm.at[wid], idx_v)
    g_sems = (gs0, gs1)
    w_sems = (ws0, ws1)

    def gather(g, b, k):
        return (x_hbm.at[idx_v.at[g * GRP + k]],
                rows_v.at[b, pl.ds(k * CHUNK, CHUNK)],
                g_sems[b])

    def write(g, b):
        return (rows_v.at[b],
                out_hbm.at[pl.ds(base + g * GROUP_ROWS, GROUP_ROWS)],
                w_sems[b])

    def outer(t, carry):
        for b in (0, 1):
            g = 2 * t + b

            @pl.when(t > 0)
            def _(b=b, g=g):
                pltpu.make_async_copy(*write(g - 2, b)).wait()

            for k in range(GRP):
                pltpu.async_copy(*gather(g, b, k))
        for b in (0, 1):
            g = 2 * t + b
            for k in range(GRP):
                pltpu.make_async_copy(*gather(g, b, k)).wait()
            pltpu.async_copy(*write(g, b))
        return carry

    lax.fori_loop(0, OUTER, outer, 0)
    for b in (0, 1):
        pltpu.make_async_copy(*write(2 * (OUTER - 1) + b, b)).wait()


@functools.cache
def _sc_gather():
    return functools.partial(
        pl.kernel,
        out_type=jax.ShapeDtypeStruct((ROWS_S, D), jnp.float32),
        mesh=plsc.VectorSubcoreMesh(core_axis_name="c", subcore_axis_name="s"),
        scratch_types=[
            pltpu.VMEM((STEPS, CHUNK), jnp.int32),
            pltpu.VMEM((2, GROUP_ROWS, D), jnp.float32),
            pltpu.SemaphoreType.DMA,
            pltpu.SemaphoreType.DMA,
            pltpu.SemaphoreType.DMA,
            pltpu.SemaphoreType.DMA,
        ],
    )(_sc_gather_body)


def _mlp_compute(xs_ref, xd_ref, ea_ref, w1a_ref, w1b_ref, w1c_ref, b1_ref,
                 w2_ref, b2_ref, out_ref):
    z = jnp.dot(xs_ref[...], w1a_ref[...], preferred_element_type=jnp.float32)
    z = z + jnp.dot(xd_ref[...], w1b_ref[...], preferred_element_type=jnp.float32)
    z = z + lax.dot_general(ea_ref[...], w1c_ref[...],
                            (((0,), (0,)), ((), ())),
                            preferred_element_type=jnp.float32)
    z = z + b1_ref[...]
    h = 0.5 * z * (1.0 + lax.erf(z * 0.7071067811865476))
    out_ref[...] = (
        jnp.dot(h, w2_ref[...], preferred_element_type=jnp.float32) + b2_ref[...]
    )


def _mlp_body_first(*refs):
    _mlp_compute(*refs)


def _mlp_body_second(prev_ref, *refs):
    del prev_ref
    _mlp_compute(*refs)


_W_SPECS = [
    pl.BlockSpec((D, D), lambda i: (0, 0)),
    pl.BlockSpec((D, D), lambda i: (0, 0)),
    pl.BlockSpec((DE, D), lambda i: (0, 0)),
    pl.BlockSpec((1, D), lambda i: (0, 0)),
    pl.BlockSpec((D, D), lambda i: (0, 0)),
    pl.BlockSpec((1, D), lambda i: (0, 0)),
]


def _mlp_slice(h, rows, edge_attr, weights, prev_out=None):
    off = h * NBLK_H
    in_specs = [
        pl.BlockSpec((BE, D), lambda i: (i, 0)),
        pl.BlockSpec((BE, D), lambda i: (i + NBLK_H, 0)),
        pl.BlockSpec((DE, BE), lambda i, off=off: (0, i + off)),
    ] + _W_SPECS
    out_spec = pl.BlockSpec((BE, D), lambda i, off=off: (i + off, 0))
    args = (rows, rows, edge_attr) + weights
    if prev_out is None:
        return pl.pallas_call(
            _mlp_body_first,
            grid=(NBLK_H,),
            in_specs=in_specs,
            out_specs=out_spec,
            out_shape=jax.ShapeDtypeStruct((E, D), jnp.float32),
        )(*args)
    return pl.pallas_call(
        _mlp_body_second,
        grid=(NBLK_H,),
        in_specs=[pl.BlockSpec(memory_space=pl.ANY)] + in_specs,
        out_specs=out_spec,
        out_shape=jax.ShapeDtypeStruct((E, D), jnp.float32),
        input_output_aliases={0: 0},
    )(prev_out, *args)


def kernel(x, edge_index, edge_attr, W1, b1, W2, b2):
    src = edge_index[0].astype(jnp.int32)
    dst = edge_index[1].astype(jnp.int32)
    edge_attr = jnp.swapaxes(edge_attr, 0, 1)
    weights = (W1[:D], W1[D:2 * D], W1[2 * D:],
               b1.reshape(1, D), W2, b2.reshape(1, D))
    sc = _sc_gather()
    rows = []
    for h in range(S):
        idx = jnp.concatenate(
            [src[h * EH:(h + 1) * EH], dst[h * EH:(h + 1) * EH]]
        ).reshape(NW, STEPS, CHUNK)
        rows.append(sc(x, idx))
    out = _mlp_slice(0, rows[0], edge_attr, weights)
    for h in range(1, S):
        out = _mlp_slice(h, rows[h], edge_attr, weights, prev_out=out)
    return out


# speedup vs baseline: 5.7011x; 1.1713x over previous
import functools

import jax
import jax.numpy as jnp
from jax import lax
from jax.experimental import pallas as pl
from jax.experimental.pallas import tpu as pltpu
from jax.experimental.pallas import tpu_sc as plsc

N_NODES = 10000
D = 128
E = 320000
DE = 16

S = 2
EH = E // S

NC = 2
NS = 16
NW = NC * NS
PER_W = EH // NW
CHUNK = 100
FE = 2 * CHUNK
FILLS = PER_W // FE
OUTER = (FILLS - 1) // 2
LANES = 16
ROW_UNROLL = 2

BN = 2000
NBLK_P = N_NODES // BN
BE = 3200
NBLK_H = EH // BE


def _pre_body(x_ref, w1a_ref, w1b_ref, out_ref):
    out_ref[0] = jnp.dot(x_ref[...], w1a_ref[...],
                         preferred_element_type=jnp.float32)
    out_ref[1] = jnp.dot(x_ref[...], w1b_ref[...],
                         preferred_element_type=jnp.float32)


def _precompute(x, W1a, W1b):
    return pl.pallas_call(
        _pre_body,
        grid=(NBLK_P,),
        in_specs=[
            pl.BlockSpec((BN, D), lambda i: (i, 0)),
            pl.BlockSpec((D, D), lambda i: (0, 0)),
            pl.BlockSpec((D, D), lambda i: (0, 0)),
        ],
        out_specs=pl.BlockSpec((2, BN, D), lambda i: (0, i, 0)),
        out_shape=jax.ShapeDtypeStruct((2, N_NODES, D), jnp.float32),
    )(x, W1a, W1b)


def _sc_gather_body(p_hbm, idx_hbm, out_hbm, idx_v, rows_v, gs0, gs1, ws0, ws1):
    c = lax.axis_index("c")
    s = lax.axis_index("s")
    wid = s * NC + c
    base = wid * PER_W
    pltpu.sync_copy(idx_hbm.at[wid], idx_v)
    g_sems = (gs0, gs1)
    w_sems = (ws0, ws1)

    def gather(q, b, half, sub):
        return (p_hbm.at[idx_v.at[q, half, sub]],
                rows_v.at[b, pl.ds((2 * half + sub) * CHUNK, CHUNK)],
                g_sems[b])

    def write(q, b):
        return (rows_v.at[b, pl.ds(0, FE)],
                out_hbm.at[pl.ds(base + q * FE, FE)],
                w_sems[b])

    def add_rows(b):
        def row_body(r, carry):
            for u in range(ROW_UNROLL):
                for l in range(D // LANES):
                    sl = pl.ds(l * LANES, LANES)
                    ri = r * ROW_UNROLL + u
                    rows_v[b, ri, sl] = (
                        rows_v[b, ri, sl] + rows_v[b, FE + ri, sl])
            return carry

        lax.fori_loop(0, FE // ROW_UNROLL, row_body, 0)

    def start_fill(q, b):
        for half in (0, 1):
            for sub in (0, 1):
                pltpu.async_copy(*gather(q, b, half, sub))

    def finish_fill(q, b):
        for half in (0, 1):
            for sub in (0, 1):
                pltpu.make_async_copy(*gather(q, b, half, sub)).wait()
        add_rows(b)
        pltpu.async_copy(*write(q, b))

    def outer(t, carry):
        for b in (0, 1):
            q = 2 * t + b

            @pl.when(t > 0)
            def _(b=b, q=q):
                pltpu.make_async_copy(*write(q - 2, b)).wait()

            start_fill(q, b)
        for b in (0, 1):
            finish_fill(2 * t + b, b)
        return carry

    lax.fori_loop(0, OUTER, outer, 0)
    q_tail = 2 * OUTER
    pltpu.make_async_copy(*write(q_tail - 2, 0)).wait()
    start_fill(q_tail, 0)
    finish_fill(q_tail, 0)
    pltpu.make_async_copy(*write(q_tail - 1, 1)).wait()
    pltpu.make_async_copy(*write(q_tail, 0)).wait()


@functools.cache
def _sc_gather():
    return functools.partial(
        pl.kernel,
        out_type=jax.ShapeDtypeStruct((EH, D), jnp.float32),
        mesh=plsc.VectorSubcoreMesh(core_axis_name="c", subcore_axis_name="s"),
        scratch_types=[
            pltpu.VMEM((FILLS, 2, 2, CHUNK), jnp.int32),
            pltpu.VMEM((2, 2 * FE, D), jnp.float32),
            pltpu.SemaphoreType.DMA,
            pltpu.SemaphoreType.DMA,
            pltpu.SemaphoreType.DMA,
            pltpu.SemaphoreType.DMA,
        ],
    )(_sc_gather_body)


def _mlp_compute(g_ref, ea_ref, w1c_ref, b1_ref, w2_ref, b2_ref, out_ref):
    z = g_ref[...] + lax.dot_general(
        ea_ref[...], w1c_ref[...], (((0,), (0,)), ((), ())),
        preferred_element_type=jnp.float32) + b1_ref[...]
    h = 0.5 * z * (1.0 + lax.erf(z * 0.7071067811865476))
    out_ref[...] = (
        jnp.dot(h, w2_ref[...], preferred_element_type=jnp.float32) + b2_ref[...]
    )


def _mlp_body_first(*refs):
    _mlp_compute(*refs)


def _mlp_body_second(prev_ref, *refs):
    del prev_ref
    _mlp_compute(*refs)


_W_SPECS = [
    pl.BlockSpec((DE, D), lambda i: (0, 0)),
    pl.BlockSpec((1, D), lambda i: (0, 0)),
    pl.BlockSpec((D, D), lambda i: (0, 0)),
    pl.BlockSpec((1, D), lambda i: (0, 0)),
]


def _mlp_slice(h, g, edge_attr_t, weights, prev_out=None):
    off = h * NBLK_H
    in_specs = [
        pl.BlockSpec((BE, D), lambda i: (i, 0)),
        pl.BlockSpec((DE, BE), lambda i, off=off: (0, i + off)),
    ] + _W_SPECS
    out_spec = pl.BlockSpec((BE, D), lambda i, off=off: (i + off, 0))
    args = (g, edge_attr_t) + weights
    if prev_out is None:
        return pl.pallas_call(
            _mlp_body_first,
            grid=(NBLK_H,),
            in_specs=in_specs,
            out_specs=out_spec,
            out_shape=jax.ShapeDtypeStruct((E, D), jnp.float32),
        )(*args)
    return pl.pallas_call(
        _mlp_body_second,
        grid=(NBLK_H,),
        in_specs=[pl.BlockSpec(memory_space=pl.ANY)] + in_specs,
        out_specs=out_spec,
        out_shape=jax.ShapeDtypeStruct((E, D), jnp.float32),
        input_output_aliases={0: 0},
    )(prev_out, *args)


def kernel(x, edge_index, edge_attr, W1, b1, W2, b2):
    src = edge_index[0].astype(jnp.int32)
    dst = edge_index[1].astype(jnp.int32)
    edge_attr_t = jnp.swapaxes(edge_attr, 0, 1)
    weights = (W1[2 * D:], b1.reshape(1, D), W2, b2.reshape(1, D))
    P = _precompute(x, W1[:D], W1[D:2 * D]).reshape(2 * N_NODES, D)
    sc = _sc_gather()
    g = []
    for h in range(S):
        s_h = src[h * EH:(h + 1) * EH].reshape(NW, FILLS, 1, 2, CHUNK)
        d_h = dst[h * EH:(h + 1) * EH].reshape(NW, FILLS, 1, 2, CHUNK) + N_NODES
        idx = jnp.concatenate([s_h, d_h], axis=2)
        g.append(sc(P, idx))
    out = _mlp_slice(0, g[0], edge_attr_t, weights)
    for h in range(1, S):
        out = _mlp_slice(h, g[h], edge_attr_t, weights, prev_out=out)
    return out
